# Initial kernel scaffold; baseline (speedup 1.0000x reference)
import functools

import jax
import jax.numpy as jnp
from jax import lax
from jax.experimental import pallas as pl
from jax.experimental.pallas import tpu as pltpu
from jax.experimental.pallas import tpu_sc as plsc

BATCH = 4096
HIST = 200
DIM = 32
B = BATCH * HIST

_info = plsc.get_sparse_core_info()
NC, NS = _info.num_cores, _info.num_subcores
NW = NC * NS
B_PER_W = B // NW
CHUNK = 512
NSTEP = B_PER_W // CHUNK


def _make_emb():
    mesh = plsc.VectorSubcoreMesh(core_axis_name="c", subcore_axis_name="s")

    @functools.partial(
        pl.kernel,
        mesh=mesh,
        out_type=jax.ShapeDtypeStruct((B, DIM), jnp.float32),
        scratch_types=[
            pltpu.VMEM((2, CHUNK), jnp.int32),
            pltpu.VMEM((2, CHUNK, DIM), jnp.float32),
            pltpu.SemaphoreType.DMA,
            pltpu.SemaphoreType.DMA,
            pltpu.SemaphoreType.DMA,
        ],
    )
    def emb(idx_hbm, table_hbm, out_hbm, idx_v, rows_v, isem, gsem, osem):
        wid = lax.axis_index("s") * NC + lax.axis_index("c")
        wbase = wid * B_PER_W

        def step(i, carry):
            for b in range(2):
                base = wbase + (2 * i + b) * CHUNK
                pltpu.async_copy(
                    idx_hbm.at[pl.ds(base, CHUNK)], idx_v.at[b], isem
                ).wait()
                pltpu.async_copy(
                    table_hbm.at[idx_v.at[b]], rows_v.at[b], gsem
                ).wait()
                pltpu.async_copy(
                    rows_v.at[b], out_hbm.at[pl.ds(base, CHUNK)], osem
                ).wait()
            return carry

        lax.fori_loop(0, NSTEP // 2, step, 0)

    return emb


_emb = _make_emb()


def kernel(tokens, table):
    idx = tokens.reshape(B).astype(jnp.int32)
    out = _emb(idx, table)
    return out.reshape(BATCH, HIST, DIM)


# speedup vs baseline: 1.4124x; 1.4124x over previous
import functools

import jax
import jax.numpy as jnp
from jax import lax
from jax.experimental import pallas as pl
from jax.experimental.pallas import tpu as pltpu
from jax.experimental.pallas import tpu_sc as plsc

BATCH = 4096
HIST = 200
DIM = 32
B = BATCH * HIST

_info = plsc.get_sparse_core_info()
NC, NS = _info.num_cores, _info.num_subcores
NW = NC * NS
B_PER_W = B // NW
CHUNK = 512
NSTEP = B_PER_W // CHUNK


def _make_emb():
    mesh = plsc.VectorSubcoreMesh(core_axis_name="c", subcore_axis_name="s")

    @functools.partial(
        pl.kernel,
        mesh=mesh,
        out_type=jax.ShapeDtypeStruct((B, DIM), jnp.float32),
        scratch_types=[
            pltpu.VMEM((2, CHUNK), jnp.int32),
            pltpu.VMEM((2, CHUNK, DIM), jnp.float32),
            pltpu.SemaphoreType.DMA,
            pltpu.SemaphoreType.DMA,
            pltpu.SemaphoreType.DMA,
        ],
        compiler_params=pltpu.CompilerParams(use_tc_tiling_on_sc=False),
    )
    def emb(idx_hbm, table_hbm, out_hbm, idx_v, rows_v, isem, gsem, osem):
        wid = lax.axis_index("s") * NC + lax.axis_index("c")
        wbase = wid * B_PER_W

        def step(i, carry):
            for b in range(2):
                base = wbase + (2 * i + b) * CHUNK
                pltpu.async_copy(
                    idx_hbm.at[pl.ds(base, CHUNK)], idx_v.at[b], isem
                ).wait()
                pltpu.async_copy(
                    table_hbm.at[idx_v.at[b]], rows_v.at[b], gsem
                ).wait()
                pltpu.async_copy(
                    rows_v.at[b], out_hbm.at[pl.ds(base, CHUNK)], osem
                ).wait()
            return carry

        lax.fori_loop(0, NSTEP // 2, step, 0)

    return emb


_emb = _make_emb()


def kernel(tokens, table):
    idx = tokens.reshape(B).astype(jnp.int32)
    out = _emb(idx, table)
    return out.reshape(BATCH, HIST, DIM)


# speedup vs baseline: 1.4986x; 1.0610x over previous
import functools

import jax
import jax.numpy as jnp
from jax import lax
from jax.experimental import pallas as pl
from jax.experimental.pallas import tpu as pltpu
from jax.experimental.pallas import tpu_sc as plsc

BATCH = 4096
HIST = 200
DIM = 32
B = BATCH * HIST

_info = plsc.get_sparse_core_info()
NC, NS = _info.num_cores, _info.num_subcores
NW = NC * NS
B_PER_W = B // NW

CHUNK = 320
NSTEP = B_PER_W // CHUNK
K = 4
NBUF = 2 * K
NG = NSTEP // K


def _make_emb():
    mesh = plsc.VectorSubcoreMesh(core_axis_name="c", subcore_axis_name="s")

    @functools.partial(
        pl.kernel,
        mesh=mesh,
        out_type=jax.ShapeDtypeStruct((B, DIM), jnp.float32),
        scratch_types=[
            pltpu.VMEM((B_PER_W,), jnp.int32),
            pltpu.VMEM((NBUF, CHUNK, DIM), jnp.float32),
            pltpu.SemaphoreType.DMA,
        ]
        + [pltpu.SemaphoreType.DMA] * (2 * NBUF),
        compiler_params=pltpu.CompilerParams(use_tc_tiling_on_sc=False),
    )
    def emb(idx_hbm, table_hbm, out_hbm, idx_all, rows, isem, *sems):
        gsems = sems[:NBUF]
        osems = sems[NBUF:]
        wid = lax.axis_index("s") * NC + lax.axis_index("c")
        wbase = wid * B_PER_W

        pltpu.async_copy(
            idx_hbm.at[pl.ds(wbase, B_PER_W)], idx_all, isem
        ).wait()

        def start_gather(i, buf):
            pltpu.async_copy(
                table_hbm.at[idx_all.at[pl.ds(i * CHUNK, CHUNK)]],
                rows.at[buf],
                gsems[buf],
            )

        def wait_gather(buf):
            pltpu.make_async_copy(
                table_hbm.at[idx_all.at[pl.ds(0, CHUNK)]],
                rows.at[buf],
                gsems[buf],
            ).wait()

        def start_store(i, buf):
            pltpu.async_copy(
                rows.at[buf],
                out_hbm.at[pl.ds(wbase + i * CHUNK, CHUNK)],
                osems[buf],
            )

        def wait_store(buf):
            pltpu.make_async_copy(
                rows.at[buf], out_hbm.at[pl.ds(wbase, CHUNK)], osems[buf]
            ).wait()

        def run_group(g, h, wait_prev_stores):
            bufs = range(h * K, (h + 1) * K)
            if wait_prev_stores:
                for buf in bufs:
                    wait_store(buf)
            for j, buf in enumerate(bufs):
                start_gather(g * K + j, buf)
            for j, buf in enumerate(bufs):
                wait_gather(buf)
                start_store(g * K + j, buf)

        run_group(0, 0, wait_prev_stores=False)
        run_group(1, 1, wait_prev_stores=False)

        def outer(jj, carry):
            g = 2 * jj + 2
            run_group(g, 0, wait_prev_stores=True)
            run_group(g + 1, 1, wait_prev_stores=True)
            return carry

        lax.fori_loop(0, (NG - 2) // 2, outer, 0)

        for buf in range(NBUF):
            wait_store(buf)

    return emb


_emb = _make_emb()


def kernel(tokens, table):
    idx = tokens.reshape(B).astype(jnp.int32)
    out = _emb(idx, table)
    return out.reshape(BATCH, HIST, DIM)
